# baseline (speedup 1.0000x reference)
import jax
import jax.numpy as jnp
from jax.experimental import pallas as pl
from jax.experimental.pallas import tpu as pltpu

_BATCH = 128
_D = 64
_N = 1_000_000
_TEMP = 0.05
_TILE = 39424


def _mm_kernel(xt_ref, ft_ref, ot_ref):
    xt = xt_ref[...]
    norm = jnp.sqrt(jnp.sum(xt * xt, axis=0, keepdims=True))
    xt = xt * ((1.0 / _TEMP) / jnp.maximum(norm, 1e-12))
    ot_ref[...] = jax.lax.dot_general(
        ft_ref[...], xt,
        dimension_numbers=(((0,), (0,)), ((), ())),
        preferred_element_type=jnp.float32,
    )


def kernel(inputs, targets, features):
    del targets
    xt = inputs.T
    ft = features.T
    out_t = pl.pallas_call(
        _mm_kernel,
        grid=(pl.cdiv(_N, _TILE),),
        in_specs=[
            pl.BlockSpec((_D, _BATCH), lambda i: (0, 0)),
            pl.BlockSpec((_D, _TILE), lambda i: (0, i)),
        ],
        out_specs=pl.BlockSpec((_TILE, _BATCH), lambda i: (i, 0)),
        out_shape=jax.ShapeDtypeStruct((_N, _BATCH), jnp.float32),
        compiler_params=pltpu.CompilerParams(
            dimension_semantics=("arbitrary",),
        ),
    )(xt, ft)
    return out_t.T


# speedup vs baseline: 1.0002x; 1.0002x over previous
import jax
import jax.numpy as jnp
from jax.experimental import pallas as pl
from jax.experimental.pallas import tpu as pltpu

_BATCH = 128
_D = 64
_N = 1_000_000
_TEMP = 0.05
_TILE = 39424


def _mm_kernel(xt_ref, ft_ref, ot_ref):
    xt = xt_ref[...]
    norm = jnp.sqrt(jnp.sum(xt * xt, axis=0, keepdims=True))
    xt = xt * ((1.0 / _TEMP) / jnp.maximum(norm, 1e-12))
    ot_ref[...] = jax.lax.dot_general(
        ft_ref[...], xt,
        dimension_numbers=(((0,), (0,)), ((), ())),
        preferred_element_type=jnp.float32,
    )


def kernel(inputs, targets, features):
    del targets
    xt = inputs.T
    ft = features.T
    out_t = pl.pallas_call(
        _mm_kernel,
        grid=(pl.cdiv(_N, _TILE),),
        in_specs=[
            pl.BlockSpec((_D, _BATCH), lambda i: (0, 0)),
            pl.BlockSpec((_D, _TILE), lambda i: (0, i)),
        ],
        out_specs=pl.BlockSpec((_TILE, _BATCH), lambda i: (i, 0)),
        out_shape=jax.ShapeDtypeStruct((_N, _BATCH), jnp.float32),
        compiler_params=pltpu.CompilerParams(
            dimension_semantics=("parallel",),
        ),
    )(xt, ft)
    return out_t.T
